# baseline (speedup 1.0000x reference)
import functools

import jax
import jax.numpy as jnp
from jax import lax
from jax.experimental import pallas as pl
from jax.experimental.pallas import tpu as pltpu
from jax.experimental.pallas import tpu_sc as plsc

EMBED = 128
BATCH = 4096
TB = 4096


@functools.lru_cache(maxsize=None)
def _make_gather(n_user, n_item, batch, dim):
    info = plsc.get_sparse_core_info()
    nw = info.num_cores * info.num_subcores
    b_per_w = batch // nw
    mesh = plsc.VectorSubcoreMesh(core_axis_name="c", subcore_axis_name="s")

    @functools.partial(
        pl.kernel,
        mesh=mesh,
        out_type=[
            jax.ShapeDtypeStruct((batch, dim), jnp.float32),
            jax.ShapeDtypeStruct((batch, dim), jnp.float32),
        ],
        scratch_types=[
            pltpu.VMEM((b_per_w,), jnp.int32),
            pltpu.VMEM((b_per_w, dim), jnp.float32),
            pltpu.VMEM((b_per_w,), jnp.int32),
            pltpu.VMEM((b_per_w, dim), jnp.float32),
            pltpu.SemaphoreType.DMA,
            pltpu.SemaphoreType.DMA,
        ],
    )
    def gather_k(ut_hbm, it_hbm, uid_hbm, iid_hbm, uout_hbm, iout_hbm,
                 uidx_v, urows_v, iidx_v, irows_v, usem, isem):
        wid = lax.axis_index("s") * info.num_cores + lax.axis_index("c")
        base = wid * b_per_w
        pltpu.sync_copy(uid_hbm.at[pl.ds(base, b_per_w)], uidx_v)
        pltpu.sync_copy(iid_hbm.at[pl.ds(base, b_per_w)], iidx_v)
        cu = pltpu.async_copy(ut_hbm.at[uidx_v], urows_v, usem)
        ci = pltpu.async_copy(it_hbm.at[iidx_v], irows_v, isem)
        cu.wait()
        ci.wait()
        pltpu.sync_copy(urows_v, uout_hbm.at[pl.ds(base, b_per_w)])
        pltpu.sync_copy(irows_v, iout_hbm.at[pl.ds(base, b_per_w)])

    return gather_k


def _mlp_body(uv_ref, iv_ref, w0_ref, b0_ref, w1_ref, b1_ref,
              w2_ref, b2_ref, o_ref):
    bf = jnp.bfloat16
    h = jnp.dot(uv_ref[...].astype(bf), w0_ref[:EMBED].astype(bf),
                preferred_element_type=jnp.float32)
    h += jnp.dot(iv_ref[...].astype(bf), w0_ref[EMBED:].astype(bf),
                 preferred_element_type=jnp.float32)
    h = jnp.maximum(h + b0_ref[...], 0.0)
    h = jnp.maximum(
        jnp.dot(h.astype(bf), w1_ref[...].astype(bf),
                preferred_element_type=jnp.float32)
        + b1_ref[...], 0.0)
    o_ref[...] = jnp.maximum(
        jnp.dot(h.astype(bf), w2_ref[...].astype(bf),
                preferred_element_type=jnp.float32)
        + b2_ref[...], 0.0)


def _mlp(uv, iv, w0, b0, w1, b1, w2, b2):
    batch = uv.shape[0]
    d_out = w2.shape[1]
    tb = min(TB, batch)
    grid = (batch // tb,)
    full = lambda a: pl.BlockSpec(a.shape, lambda i: (0,) * a.ndim)
    return pl.pallas_call(
        _mlp_body,
        grid=grid,
        in_specs=[
            pl.BlockSpec((tb, uv.shape[1]), lambda i: (i, 0)),
            pl.BlockSpec((tb, iv.shape[1]), lambda i: (i, 0)),
            full(w0), full(b0), full(w1), full(b1), full(w2), full(b2),
        ],
        out_specs=pl.BlockSpec((tb, d_out), lambda i: (i, 0)),
        out_shape=jax.ShapeDtypeStruct((batch, d_out), jnp.float32),
    )(uv, iv, w0, b0, w1, b1, w2, b2)


N_CHUNKS = 1


def kernel(user_id, item_id, user_table, item_table, W0, b0, W1, b1, W2, b2):
    batch = user_id.shape[0]
    cb = batch // N_CHUNKS
    gather = _make_gather(user_table.shape[0], item_table.shape[0],
                          cb, user_table.shape[1])
    b0r, b1r, b2r = b0.reshape(1, -1), b1.reshape(1, -1), b2.reshape(1, -1)
    outs = []
    for c in range(N_CHUNKS):
        uv, iv = gather(user_table, item_table,
                        user_id[c * cb:(c + 1) * cb],
                        item_id[c * cb:(c + 1) * cb])
        outs.append(_mlp(uv, iv, W0, b0r, W1, b1r, W2, b2r))
    return jnp.concatenate(outs, axis=0) if N_CHUNKS > 1 else outs[0]


# speedup vs baseline: 1.0029x; 1.0029x over previous
import functools

import jax
import jax.numpy as jnp
from jax import lax
from jax.experimental import pallas as pl
from jax.experimental.pallas import tpu as pltpu
from jax.experimental.pallas import tpu_sc as plsc

EMBED = 128
BATCH = 4096
TB = 2048


@functools.lru_cache(maxsize=None)
def _make_gather(n_user, n_item, batch, dim):
    info = plsc.get_sparse_core_info()
    nw = info.num_cores * info.num_subcores
    b_per_w = batch // nw
    mesh = plsc.VectorSubcoreMesh(core_axis_name="c", subcore_axis_name="s")

    @functools.partial(
        pl.kernel,
        mesh=mesh,
        out_type=[
            jax.ShapeDtypeStruct((batch, dim), jnp.float32),
            jax.ShapeDtypeStruct((batch, dim), jnp.float32),
        ],
        scratch_types=[
            pltpu.VMEM((b_per_w,), jnp.int32),
            pltpu.VMEM((b_per_w, dim), jnp.float32),
            pltpu.VMEM((b_per_w,), jnp.int32),
            pltpu.VMEM((b_per_w, dim), jnp.float32),
            pltpu.SemaphoreType.DMA,
            pltpu.SemaphoreType.DMA,
        ],
    )
    def gather_k(ut_hbm, it_hbm, uid_hbm, iid_hbm, uout_hbm, iout_hbm,
                 uidx_v, urows_v, iidx_v, irows_v, usem, isem):
        wid = lax.axis_index("s") * info.num_cores + lax.axis_index("c")
        base = wid * b_per_w
        pltpu.sync_copy(uid_hbm.at[pl.ds(base, b_per_w)], uidx_v)
        pltpu.sync_copy(iid_hbm.at[pl.ds(base, b_per_w)], iidx_v)
        cu = pltpu.async_copy(ut_hbm.at[uidx_v], urows_v, usem)
        ci = pltpu.async_copy(it_hbm.at[iidx_v], irows_v, isem)
        cu.wait()
        ci.wait()
        pltpu.sync_copy(urows_v, uout_hbm.at[pl.ds(base, b_per_w)])
        pltpu.sync_copy(irows_v, iout_hbm.at[pl.ds(base, b_per_w)])

    return gather_k


def _mlp_body(uv_ref, iv_ref, w0_ref, b0_ref, w1_ref, b1_ref,
              w2_ref, b2_ref, o_ref):
    bf = jnp.bfloat16
    h = jnp.dot(uv_ref[...].astype(bf), w0_ref[:EMBED].astype(bf),
                preferred_element_type=jnp.float32)
    h += jnp.dot(iv_ref[...].astype(bf), w0_ref[EMBED:].astype(bf),
                 preferred_element_type=jnp.float32)
    h = jnp.maximum(h + b0_ref[...], 0.0)
    h = jnp.maximum(
        jnp.dot(h.astype(bf), w1_ref[...].astype(bf),
                preferred_element_type=jnp.float32)
        + b1_ref[...], 0.0)
    o_ref[...] = jnp.maximum(
        jnp.dot(h.astype(bf), w2_ref[...].astype(bf),
                preferred_element_type=jnp.float32)
        + b2_ref[...], 0.0)


def _mlp(uv, iv, w0, b0, w1, b1, w2, b2):
    batch = uv.shape[0]
    d_out = w2.shape[1]
    tb = min(TB, batch)
    grid = (batch // tb,)
    full = lambda a: pl.BlockSpec(a.shape, lambda i: (0,) * a.ndim)
    return pl.pallas_call(
        _mlp_body,
        grid=grid,
        in_specs=[
            pl.BlockSpec((tb, uv.shape[1]), lambda i: (i, 0)),
            pl.BlockSpec((tb, iv.shape[1]), lambda i: (i, 0)),
            full(w0), full(b0), full(w1), full(b1), full(w2), full(b2),
        ],
        out_specs=pl.BlockSpec((tb, d_out), lambda i: (i, 0)),
        out_shape=jax.ShapeDtypeStruct((batch, d_out), jnp.float32),
    )(uv, iv, w0, b0, w1, b1, w2, b2)


N_CHUNKS = 1


def kernel(user_id, item_id, user_table, item_table, W0, b0, W1, b1, W2, b2):
    batch = user_id.shape[0]
    cb = batch // N_CHUNKS
    gather = _make_gather(user_table.shape[0], item_table.shape[0],
                          cb, user_table.shape[1])
    b0r, b1r, b2r = b0.reshape(1, -1), b1.reshape(1, -1), b2.reshape(1, -1)
    outs = []
    for c in range(N_CHUNKS):
        uv, iv = gather(user_table, item_table,
                        user_id[c * cb:(c + 1) * cb],
                        item_id[c * cb:(c + 1) * cb])
        outs.append(_mlp(uv, iv, W0, b0r, W1, b1r, W2, b2r))
    return jnp.concatenate(outs, axis=0) if N_CHUNKS > 1 else outs[0]


# speedup vs baseline: 1.0131x; 1.0102x over previous
import functools

import jax
import jax.numpy as jnp
from jax import lax
from jax.experimental import pallas as pl
from jax.experimental.pallas import tpu as pltpu
from jax.experimental.pallas import tpu_sc as plsc

EMBED = 128
BATCH = 4096
TB = 2048


@functools.lru_cache(maxsize=None)
def _make_gather(n_user, n_item, batch, dim):
    info = plsc.get_sparse_core_info()
    nw = info.num_cores * info.num_subcores
    b_per_w = batch // nw
    mesh = plsc.VectorSubcoreMesh(core_axis_name="c", subcore_axis_name="s")

    @functools.partial(
        pl.kernel,
        mesh=mesh,
        out_type=[
            jax.ShapeDtypeStruct((batch, dim), jnp.float32),
            jax.ShapeDtypeStruct((batch, dim), jnp.float32),
        ],
        scratch_types=[
            pltpu.VMEM((b_per_w,), jnp.int32),
            pltpu.VMEM((b_per_w, dim), jnp.float32),
            pltpu.VMEM((b_per_w,), jnp.int32),
            pltpu.VMEM((b_per_w, dim), jnp.float32),
            pltpu.SemaphoreType.DMA,
            pltpu.SemaphoreType.DMA,
            pltpu.SemaphoreType.DMA,
            pltpu.SemaphoreType.DMA,
        ],
    )
    def gather_k(ut_hbm, it_hbm, uid_hbm, iid_hbm, uout_hbm, iout_hbm,
                 uidx_v, urows_v, iidx_v, irows_v, usem, isem, wusem, wisem):
        wid = lax.axis_index("s") * info.num_cores + lax.axis_index("c")
        base = wid * b_per_w
        pltpu.sync_copy(uid_hbm.at[pl.ds(base, b_per_w)], uidx_v)
        pltpu.sync_copy(iid_hbm.at[pl.ds(base, b_per_w)], iidx_v)
        cu = pltpu.async_copy(ut_hbm.at[uidx_v], urows_v, usem)
        ci = pltpu.async_copy(it_hbm.at[iidx_v], irows_v, isem)
        cu.wait()
        wu = pltpu.async_copy(urows_v, uout_hbm.at[pl.ds(base, b_per_w)], wusem)
        ci.wait()
        wi = pltpu.async_copy(irows_v, iout_hbm.at[pl.ds(base, b_per_w)], wisem)
        wu.wait()
        wi.wait()

    return gather_k


def _mlp_body(uv_ref, iv_ref, w0_ref, b0_ref, w1_ref, b1_ref,
              w2_ref, b2_ref, o_ref):
    bf = jnp.bfloat16
    h = jnp.dot(uv_ref[...].astype(bf), w0_ref[:EMBED].astype(bf),
                preferred_element_type=jnp.float32)
    h += jnp.dot(iv_ref[...].astype(bf), w0_ref[EMBED:].astype(bf),
                 preferred_element_type=jnp.float32)
    h = jnp.maximum(h + b0_ref[...], 0.0)
    h = jnp.maximum(
        jnp.dot(h.astype(bf), w1_ref[...].astype(bf),
                preferred_element_type=jnp.float32)
        + b1_ref[...], 0.0)
    o_ref[...] = jnp.maximum(
        jnp.dot(h.astype(bf), w2_ref[...].astype(bf),
                preferred_element_type=jnp.float32)
        + b2_ref[...], 0.0)


def _mlp(uv, iv, w0, b0, w1, b1, w2, b2):
    batch = uv.shape[0]
    d_out = w2.shape[1]
    tb = min(TB, batch)
    grid = (batch // tb,)
    full = lambda a: pl.BlockSpec(a.shape, lambda i: (0,) * a.ndim)
    return pl.pallas_call(
        _mlp_body,
        grid=grid,
        in_specs=[
            pl.BlockSpec((tb, uv.shape[1]), lambda i: (i, 0)),
            pl.BlockSpec((tb, iv.shape[1]), lambda i: (i, 0)),
            full(w0), full(b0), full(w1), full(b1), full(w2), full(b2),
        ],
        out_specs=pl.BlockSpec((tb, d_out), lambda i: (i, 0)),
        out_shape=jax.ShapeDtypeStruct((batch, d_out), jnp.float32),
    )(uv, iv, w0, b0, w1, b1, w2, b2)


N_CHUNKS = 1


def kernel(user_id, item_id, user_table, item_table, W0, b0, W1, b1, W2, b2):
    batch = user_id.shape[0]
    cb = batch // N_CHUNKS
    gather = _make_gather(user_table.shape[0], item_table.shape[0],
                          cb, user_table.shape[1])
    b0r, b1r, b2r = b0.reshape(1, -1), b1.reshape(1, -1), b2.reshape(1, -1)
    outs = []
    for c in range(N_CHUNKS):
        uv, iv = gather(user_table, item_table,
                        user_id[c * cb:(c + 1) * cb],
                        item_id[c * cb:(c + 1) * cb])
        outs.append(_mlp(uv, iv, W0, b0r, W1, b1r, W2, b2r))
    return jnp.concatenate(outs, axis=0) if N_CHUNKS > 1 else outs[0]
